# baseline (speedup 1.0000x reference)
import jax
import jax.numpy as jnp
from jax import lax
from jax.experimental import pallas as pl
from jax.experimental.pallas import tpu as pltpu

_N = 32768
_G = 16
_R = 256
_C = 128
_F32 = jnp.float32


def _mm(a, b):
    return lax.dot_general(
        a, b, (((1,), (0,)), ((), ())),
        preferred_element_type=_F32, precision=lax.Precision.HIGHEST)


def _shift_next(x, rowsel_last):
    a = pltpu.roll(x, shift=_C - 1, axis=1)
    b = pltpu.roll(a, shift=_R - 1, axis=0)
    return jnp.where(rowsel_last, b, a)


def _shift_prev(x, rowsel_first):
    a = pltpu.roll(x, shift=1, axis=1)
    b = pltpu.roll(a, shift=1, axis=0)
    return jnp.where(rowsel_first, b, a)


def _cumsum3_flat(v1, v2, v3, u_tri, s_tri):
    v = jnp.concatenate([v1, v2, v3], axis=0)
    rowcum = _mm(v, u_tri)
    rc = [rowcum[i * _R:(i + 1) * _R] for i in range(3)]
    rowtot = jnp.concatenate(
        [jnp.broadcast_to(r[:, _C - 1:_C], (_R, _C)) for r in rc], axis=1)
    prefix = _mm(s_tri, rowtot)
    return [rc[i] + prefix[:, i * _C:(i + 1) * _C] for i in range(3)]


def _cummax_flat(v, colid, rowid):
    x = v
    s = 1
    while s < _C:
        sh = pltpu.roll(x, shift=s, axis=1)
        x = jnp.maximum(x, jnp.where(colid >= s, sh, 0.0))
        s *= 2
    z = jnp.broadcast_to(x[:, _C - 1:_C], (_R, _C))
    z = jnp.where(rowid >= 1, pltpu.roll(z, shift=1, axis=0), 0.0)
    s = 1
    while s < _R:
        sh = pltpu.roll(z, shift=s, axis=0)
        z = jnp.maximum(z, jnp.where(rowid >= s, sh, 0.0))
        s *= 2
    return jnp.maximum(x, z)


def _body(pos_ref, t_ref, b_ref,
          x0, x1, x2, x3, x4, x5,
          e0, e1, e2, e3, e4, e5, s_ref):
    pos2 = pos_ref[:, :]
    t = t_ref[:, :]
    b = b_ref[:, :]

    colid = lax.broadcasted_iota(jnp.int32, (_R, _C), 1)
    rowid = lax.broadcasted_iota(jnp.int32, (_R, _C), 0)
    rowsel_last = colid == (_C - 1)
    rowsel_first = colid == 0

    u_tri = (lax.broadcasted_iota(jnp.int32, (_C, _C), 0)
             <= lax.broadcasted_iota(jnp.int32, (_C, _C), 1)).astype(_F32)
    s_tri = (lax.broadcasted_iota(jnp.int32, (_R, _R), 1)
             < lax.broadcasted_iota(jnp.int32, (_R, _R), 0)).astype(_F32)

    k2 = lax.broadcasted_iota(jnp.int32, (2 * _C, _C), 0)
    c2 = lax.broadcasted_iota(jnp.int32, (2 * _C, _C), 1)
    px = _mm(pos2, (k2 == 2 * c2).astype(_F32))
    py = _mm(pos2, (k2 == 2 * c2 + 1).astype(_F32))

    b_next = _shift_next(b, rowsel_last)
    b_prev = _shift_prev(b, rowsel_first)
    last = b != b_next
    first = b != b_prev
    nlf = jnp.where(last, 0.0, 1.0)

    px_next = _shift_next(px, rowsel_last)
    py_next = _shift_next(py, rowsel_last)
    t_next = _shift_next(t, rowsel_last)

    ex = px_next - px
    ey = py_next - py
    drx = ex * nlf
    dry = ey * nlf
    dr2 = 1e-05 + drx * drx + dry * dry
    dr_norm = jnp.sqrt(dr2)
    dr4 = dr2 * dr2
    dto = jnp.sqrt(px * px + py * py + 1e-07)

    cs_d, cs_s, cs_q = _cumsum3_flat(dr_norm, dr2, dr4, u_tri, s_tri)
    cm_dto = _cummax_flat(dto, colid, rowid)

    ecs_d = cs_d - dr_norm
    ecs_s = cs_s - dr2
    ecs_q = cs_q - dr4
    tot_d = jnp.sum(cs_d[_R - 1:_R, _C - 1:_C])
    tot_s = jnp.sum(cs_s[_R - 1:_R, _C - 1:_C])
    tot_q = jnp.sum(cs_q[_R - 1:_R, _C - 1:_C])
    sqrt_eps = jnp.sqrt(jnp.float32(1e-05))

    inv_dur_n = jnp.zeros((_R, _C), _F32)
    inv_td_n = jnp.zeros((_R, _C), _F32)
    inv_ts_n = jnp.zeros((_R, _C), _F32)
    inv_tq_n = jnp.zeros((_R, _C), _F32)
    inv_ps_n = jnp.zeros((_R, _C), _F32)
    off_d_n = jnp.zeros((_R, _C), _F32)
    off_s_n = jnp.zeros((_R, _C), _F32)
    off_q_n = jnp.zeros((_R, _C), _F32)
    fdto_n = jnp.zeros((_R, _C), _F32)

    rowid16 = lax.broadcasted_iota(jnp.int32, (_G, _C), 0)
    colid16 = lax.broadcasted_iota(jnp.int32, (_G, _C), 1)
    s_acc = jnp.zeros((_G, _C), _F32)

    offs_d, offs_s, offs_q = [], [], []
    mfs, fdtos, cnts, durs = [], [], [], []
    spxs, spys, spx2s, spy2s = [], [], [], []
    for g in range(_G):
        mf = jnp.where(b == g, 1.0, 0.0)
        fmf = jnp.where(jnp.logical_and(first, b == g), 1.0, 0.0)
        mfs.append(mf)
        cnts.append(jnp.sum(mf))
        spxs.append(jnp.sum(mf * px))
        spys.append(jnp.sum(mf * py))
        spx2s.append(jnp.sum(mf * (px * px)))
        spy2s.append(jnp.sum(mf * (py * py)))
        durs.append(jnp.max(mf * t))
        offs_d.append(jnp.sum(fmf * ecs_d))
        offs_s.append(jnp.sum(fmf * ecs_s))
        offs_q.append(jnp.sum(fmf * ecs_q))
        fdtos.append(jnp.sum(fmf * dto))

    offs_d.append(tot_d)
    offs_s.append(tot_s)
    offs_q.append(tot_q)

    for g in range(_G):
        mf = mfs[g]
        cnt = cnts[g]
        dur_g = durs[g]
        td_g = offs_d[g + 1] - offs_d[g]
        ts_g = offs_s[g + 1] - offs_s[g]
        tq_g = offs_q[g + 1] - offs_q[g]
        ss_g = td_g - sqrt_eps
        sv2_g = ts_g - 1e-05

        inv_cnt = 1.0 / cnt
        mean_x = spxs[g] * inv_cnt
        mean_y = spys[g] * inv_cnt
        var_x = jnp.maximum(spx2s[g] * inv_cnt - mean_x * mean_x, 0.0)
        var_y = jnp.maximum(spy2s[g] * inv_cnt - mean_y * mean_y, 0.0)
        pos_std = jnp.sqrt(var_x + var_y + 1e-12)
        inv_cnt_in = 1.0 / (cnt - 1.0)
        step_mean = ss_g * inv_cnt_in
        step_var = sv2_g * inv_cnt_in
        step_std = jnp.sqrt(jnp.maximum(step_var - step_mean * step_mean, 0.0))
        mean_time_step = dur_g * inv_cnt

        inv_dur_n = inv_dur_n + mf * (1.0 / dur_g)
        inv_td_n = inv_td_n + mf * (1.0 / (td_g + 1e-07))
        inv_ts_n = inv_ts_n + mf * (1.0 / (ts_g + 1e-07))
        inv_tq_n = inv_tq_n + mf * (1.0 / (tq_g + 1e-07))
        inv_ps_n = inv_ps_n + mf * (1.0 / (pos_std + 1e-07))
        off_d_n = off_d_n + mf * offs_d[g]
        off_s_n = off_s_n + mf * offs_s[g]
        off_q_n = off_q_n + mf * offs_q[g]
        fdto_n = fdto_n + mf * fdtos[g]

        rsel = (rowid16 == g).astype(_F32)
        scale_row = (pos_std * (colid16 == 0) + ss_g * (colid16 == 1)
                     + step_std * (colid16 == 2) + step_mean * (colid16 == 3)
                     + step_var * (colid16 == 4)
                     + mean_time_step * (colid16 == 5)).astype(_F32)
        s_acc = s_acc + rsel * scale_row

    time_norm = t * inv_dur_n
    cum_d = cs_d - off_d_n
    cum_s = cs_s - off_s_n
    cum_q = cs_q - off_q_n

    x0[:, :] = time_norm
    x1[:, :] = cum_d * inv_td_n
    x2[:, :] = cum_s * inv_ts_n
    x3[:, :] = cum_q * inv_tq_n
    x4[:, :] = dto * inv_ps_n
    x5[:, :] = (cm_dto + fdto_n) * inv_ps_n

    tn_next = _shift_next(time_norm, rowsel_last)
    td = t_next - t
    d_edge = jnp.sqrt(ex * ex + ey * ey + 1e-07)
    inv_abs_td = 1.0 / (jnp.abs(td) + 1e-07)

    e0[:, :] = td
    e1[:, :] = tn_next - time_norm
    e2[:, :] = d_edge * inv_abs_td
    e3[:, :] = (_shift_next(cum_d, rowsel_last) - cum_d) * inv_abs_td
    e4[:, :] = (_shift_next(cum_s, rowsel_last) - cum_s) * inv_abs_td
    e5[:, :] = (_shift_next(cum_q, rowsel_last) - cum_q) * inv_abs_td
    s_ref[:, :] = s_acc[:, :6]


def kernel(pos, time, batch, source, target):
    del source, target
    pos2 = pos.astype(_F32).reshape(_R, 2 * _C)
    t2 = time.reshape(_R, _C).astype(_F32)
    b2 = batch.reshape(_R, _C).astype(jnp.int32)

    blk = jax.ShapeDtypeStruct((_R, _C), _F32)
    outs = pl.pallas_call(
        _body,
        out_shape=[blk] * 12 + [jax.ShapeDtypeStruct((_G, 6), _F32)],
    )(pos2, t2, b2)

    X = jnp.stack([o.reshape(_N) for o in outs[:6]], axis=1)
    E = jnp.stack([o.reshape(_N) for o in outs[6:12]], axis=1)
    return X, E, outs[12]


# speedup vs baseline: 2.1507x; 2.1507x over previous
import jax
import jax.numpy as jnp
from jax import lax
from jax.experimental import pallas as pl
from jax.experimental.pallas import tpu as pltpu

_N = 32768
_G = 16
_R = 256
_C = 128
_F32 = jnp.float32


def _mm(a, b):
    return lax.dot_general(
        a, b, (((1,), (0,)), ((), ())),
        preferred_element_type=_F32, precision=lax.Precision.HIGHEST)


def _shift_next(x, rowsel_last):
    a = pltpu.roll(x, shift=_C - 1, axis=1)
    b = pltpu.roll(a, shift=_R - 1, axis=0)
    return jnp.where(rowsel_last, b, a)


def _shift_prev(x, rowsel_first):
    a = pltpu.roll(x, shift=1, axis=1)
    b = pltpu.roll(a, shift=1, axis=0)
    return jnp.where(rowsel_first, b, a)


def _cumsum3_flat(v1, v2, v3, u_tri, s_tri):
    v = jnp.concatenate([v1, v2, v3], axis=0)
    rowcum = _mm(v, u_tri)
    rc = [rowcum[i * _R:(i + 1) * _R] for i in range(3)]
    rowtot = jnp.concatenate(
        [jnp.broadcast_to(r[:, _C - 1:_C], (_R, _C)) for r in rc], axis=1)
    prefix = _mm(s_tri, rowtot)
    return [rc[i] + prefix[:, i * _C:(i + 1) * _C] for i in range(3)]


def _cummax_flat(v, colid, rowid):
    x = v
    s = 1
    while s < _C:
        sh = pltpu.roll(x, shift=s, axis=1)
        x = jnp.maximum(x, jnp.where(colid >= s, sh, 0.0))
        s *= 2
    z = jnp.broadcast_to(x[:, _C - 1:_C], (_R, _C))
    z = jnp.where(rowid >= 1, pltpu.roll(z, shift=1, axis=0), 0.0)
    s = 1
    while s < _R:
        sh = pltpu.roll(z, shift=s, axis=0)
        z = jnp.maximum(z, jnp.where(rowid >= s, sh, 0.0))
        s *= 2
    return jnp.maximum(x, z)


def _body(px_ref, py_ref, t_ref, b_ref,
          x0, x1, x2, x3, x4, x5,
          e0, e1, e2, e3, e4, e5, s_ref):
    px = px_ref[:, :]
    py = py_ref[:, :]
    t = t_ref[:, :]
    b = b_ref[:, :]

    colid = lax.broadcasted_iota(jnp.int32, (_R, _C), 1)
    rowid = lax.broadcasted_iota(jnp.int32, (_R, _C), 0)
    rowsel_last = colid == (_C - 1)
    rowsel_first = colid == 0

    u_tri = (lax.broadcasted_iota(jnp.int32, (_C, _C), 0)
             <= lax.broadcasted_iota(jnp.int32, (_C, _C), 1)).astype(_F32)
    s_tri = (lax.broadcasted_iota(jnp.int32, (_R, _R), 1)
             < lax.broadcasted_iota(jnp.int32, (_R, _R), 0)).astype(_F32)

    b_next = _shift_next(b, rowsel_last)
    b_prev = _shift_prev(b, rowsel_first)
    last = b != b_next
    first = b != b_prev
    nlf = jnp.where(last, 0.0, 1.0)

    px_next = _shift_next(px, rowsel_last)
    py_next = _shift_next(py, rowsel_last)
    t_next = _shift_next(t, rowsel_last)

    ex = px_next - px
    ey = py_next - py
    drx = ex * nlf
    dry = ey * nlf
    dr2 = 1e-05 + drx * drx + dry * dry
    dr_norm = jnp.sqrt(dr2)
    dr4 = dr2 * dr2
    dto = jnp.sqrt(px * px + py * py + 1e-07)

    cs_d, cs_s, cs_q = _cumsum3_flat(dr_norm, dr2, dr4, u_tri, s_tri)
    cm_dto = _cummax_flat(dto, colid, rowid)

    ecs_d = cs_d - dr_norm
    ecs_s = cs_s - dr2
    ecs_q = cs_q - dr4
    tot_d = jnp.sum(cs_d[_R - 1:_R, _C - 1:_C])
    tot_s = jnp.sum(cs_s[_R - 1:_R, _C - 1:_C])
    tot_q = jnp.sum(cs_q[_R - 1:_R, _C - 1:_C])
    sqrt_eps = jnp.sqrt(jnp.float32(1e-05))

    inv_dur_n = jnp.zeros((_R, _C), _F32)
    inv_td_n = jnp.zeros((_R, _C), _F32)
    inv_ts_n = jnp.zeros((_R, _C), _F32)
    inv_tq_n = jnp.zeros((_R, _C), _F32)
    inv_ps_n = jnp.zeros((_R, _C), _F32)
    off_d_n = jnp.zeros((_R, _C), _F32)
    off_s_n = jnp.zeros((_R, _C), _F32)
    off_q_n = jnp.zeros((_R, _C), _F32)
    fdto_n = jnp.zeros((_R, _C), _F32)

    rowid16 = lax.broadcasted_iota(jnp.int32, (_G, _C), 0)
    colid16 = lax.broadcasted_iota(jnp.int32, (_G, _C), 1)
    s_acc = jnp.zeros((_G, _C), _F32)

    offs_d, offs_s, offs_q = [], [], []
    mfs, fdtos, cnts, durs = [], [], [], []
    spxs, spys, spx2s, spy2s = [], [], [], []
    for g in range(_G):
        mf = jnp.where(b == g, 1.0, 0.0)
        fmf = jnp.where(jnp.logical_and(first, b == g), 1.0, 0.0)
        mfs.append(mf)
        cnts.append(jnp.sum(mf))
        spxs.append(jnp.sum(mf * px))
        spys.append(jnp.sum(mf * py))
        spx2s.append(jnp.sum(mf * (px * px)))
        spy2s.append(jnp.sum(mf * (py * py)))
        durs.append(jnp.max(mf * t))
        offs_d.append(jnp.sum(fmf * ecs_d))
        offs_s.append(jnp.sum(fmf * ecs_s))
        offs_q.append(jnp.sum(fmf * ecs_q))
        fdtos.append(jnp.sum(fmf * dto))

    offs_d.append(tot_d)
    offs_s.append(tot_s)
    offs_q.append(tot_q)

    for g in range(_G):
        mf = mfs[g]
        cnt = cnts[g]
        dur_g = durs[g]
        td_g = offs_d[g + 1] - offs_d[g]
        ts_g = offs_s[g + 1] - offs_s[g]
        tq_g = offs_q[g + 1] - offs_q[g]
        ss_g = td_g - sqrt_eps
        sv2_g = ts_g - 1e-05

        inv_cnt = 1.0 / cnt
        mean_x = spxs[g] * inv_cnt
        mean_y = spys[g] * inv_cnt
        var_x = jnp.maximum(spx2s[g] * inv_cnt - mean_x * mean_x, 0.0)
        var_y = jnp.maximum(spy2s[g] * inv_cnt - mean_y * mean_y, 0.0)
        pos_std = jnp.sqrt(var_x + var_y + 1e-12)
        inv_cnt_in = 1.0 / (cnt - 1.0)
        step_mean = ss_g * inv_cnt_in
        step_var = sv2_g * inv_cnt_in
        step_std = jnp.sqrt(jnp.maximum(step_var - step_mean * step_mean, 0.0))
        mean_time_step = dur_g * inv_cnt

        inv_dur_n = inv_dur_n + mf * (1.0 / dur_g)
        inv_td_n = inv_td_n + mf * (1.0 / (td_g + 1e-07))
        inv_ts_n = inv_ts_n + mf * (1.0 / (ts_g + 1e-07))
        inv_tq_n = inv_tq_n + mf * (1.0 / (tq_g + 1e-07))
        inv_ps_n = inv_ps_n + mf * (1.0 / (pos_std + 1e-07))
        off_d_n = off_d_n + mf * offs_d[g]
        off_s_n = off_s_n + mf * offs_s[g]
        off_q_n = off_q_n + mf * offs_q[g]
        fdto_n = fdto_n + mf * fdtos[g]

        rsel = (rowid16 == g).astype(_F32)
        scale_row = (pos_std * (colid16 == 0) + ss_g * (colid16 == 1)
                     + step_std * (colid16 == 2) + step_mean * (colid16 == 3)
                     + step_var * (colid16 == 4)
                     + mean_time_step * (colid16 == 5)).astype(_F32)
        s_acc = s_acc + rsel * scale_row

    time_norm = t * inv_dur_n
    cum_d = cs_d - off_d_n
    cum_s = cs_s - off_s_n
    cum_q = cs_q - off_q_n

    x0[:, :] = time_norm
    x1[:, :] = cum_d * inv_td_n
    x2[:, :] = cum_s * inv_ts_n
    x3[:, :] = cum_q * inv_tq_n
    x4[:, :] = dto * inv_ps_n
    x5[:, :] = (cm_dto + fdto_n) * inv_ps_n

    tn_next = _shift_next(time_norm, rowsel_last)
    td = t_next - t
    d_edge = jnp.sqrt(ex * ex + ey * ey + 1e-07)
    inv_abs_td = 1.0 / (jnp.abs(td) + 1e-07)

    e0[:, :] = td
    e1[:, :] = tn_next - time_norm
    e2[:, :] = d_edge * inv_abs_td
    e3[:, :] = (_shift_next(cum_d, rowsel_last) - cum_d) * inv_abs_td
    e4[:, :] = (_shift_next(cum_s, rowsel_last) - cum_s) * inv_abs_td
    e5[:, :] = (_shift_next(cum_q, rowsel_last) - cum_q) * inv_abs_td
    s_ref[:, :] = s_acc[:, :6]


def kernel(pos, time, batch, source, target):
    del source, target
    px = pos[:, 0].reshape(_R, _C).astype(_F32)
    py = pos[:, 1].reshape(_R, _C).astype(_F32)
    t2 = time.reshape(_R, _C).astype(_F32)
    b2 = batch.reshape(_R, _C).astype(jnp.int32)

    blk = jax.ShapeDtypeStruct((_R, _C), _F32)
    outs = pl.pallas_call(
        _body,
        out_shape=[blk] * 12 + [jax.ShapeDtypeStruct((_G, 6), _F32)],
    )(px, py, t2, b2)

    X = jnp.stack([o.reshape(_N) for o in outs[:6]], axis=1)
    E = jnp.stack([o.reshape(_N) for o in outs[6:12]], axis=1)
    return X, E, outs[12]
